# baseline (speedup 1.0000x reference)
import functools

import jax
import jax.numpy as jnp
from jax import lax
from jax.experimental import pallas as pl
from jax.experimental.pallas import tpu as pltpu
from jax.experimental.pallas import tpu_sc as plsc

_B = 8
_S = 2048
_V = 2048
_D = 64
_K = 1000
_TOK = 32000


def _sc_hist_body(ids_hbm, out_hbm, idx_v, hist_v):
    c = lax.axis_index("c")
    s = lax.axis_index("s")
    b = s * 2 + c

    @pl.when(b < _B)
    def _():
        pltpu.sync_copy(ids_hbm.at[b], idx_v)
        zeros16 = jnp.zeros((16,), jnp.float32)

        def zbody(i, _):
            base = i * 256
            for j in range(16):
                hist_v[pl.ds(base + j * 16, 16)] = zeros16
            return 0

        lax.fori_loop(0, _TOK // 256, zbody, 0)
        ones16 = jnp.ones((16,), jnp.float32)

        def sbody(k, _):
            base = k * 64
            for j in range(4):
                ids16 = idx_v[pl.ds(base + j * 16, 16)]
                plsc.addupdate_scatter(hist_v, [ids16], ones16)
            return 0

        lax.fori_loop(0, _S // 64, sbody, 0)
        pltpu.sync_copy(hist_v, out_hbm.at[b])


@functools.cache
def _sc_hist():
    return pl.kernel(
        _sc_hist_body,
        out_type=jax.ShapeDtypeStruct((_B, _TOK), jnp.float32),
        mesh=plsc.VectorSubcoreMesh(core_axis_name="c", subcore_axis_name="s"),
        scratch_types=[
            pltpu.VMEM((_S,), jnp.int32),
            pltpu.VMEM((_TOK,), jnp.float32),
        ],
        compiler_params=pltpu.CompilerParams(needs_layout_passes=False,
                                             skip_device_barrier=True),
    )


def _lane_count(mask_bool, ones_col):
    return lax.dot_general(
        mask_bool.astype(jnp.bfloat16), ones_col,
        (((1,), (0,)), ((), ())),
        preferred_element_type=jnp.float32,
    )


def _tc_head_body(counts_ref, embt_ref, mask_ref, w_ref, b_ref, gum_ref,
                  values_ref, logprobs_ref, actions_ref):
    psum = lax.dot_general(
        counts_ref[...].astype(jnp.bfloat16), embt_ref[...],
        (((1,), (1,)), ((), ())),
        preferred_element_type=jnp.float32,
    )
    mask = mask_ref[...].astype(jnp.float32)
    denom = jnp.maximum(jnp.sum(mask, axis=1, keepdims=True), 1e-6)
    pooled = psum / denom

    logits = jnp.dot(pooled, w_ref[...],
                     preferred_element_type=jnp.float32,
                     precision=lax.Precision.HIGHEST) + b_ref[...]

    rowmax = jnp.max(logits, axis=1, keepdims=True)
    values_ref[...] = jnp.transpose(jax.nn.sigmoid(rowmax))

    shifted = logits - rowmax
    lse = jnp.log(jnp.sum(jnp.exp(shifted), axis=1, keepdims=True))
    logp = shifted - lse

    g0 = logits + gum_ref[...]
    s = lax.bitcast_convert_type(g0, jnp.int32)
    skey = jnp.where(s >= 0, s, s ^ jnp.int32(0x7FFFFFFF))

    ones_col = jnp.ones((_V, 1), jnp.bfloat16)

    t0 = jnp.full((_B, 1), jnp.int32(-2147483648))
    kf = jnp.float32(_K)

    def vbody(i, t):
        bit = (jnp.int32(28) - 4 * i).astype(jnp.int32)
        step = lax.shift_left(jnp.int32(1), bit)
        q = jnp.zeros((_B, 1), jnp.int32)
        for m in range(1, 16):
            cnt = _lane_count(skey >= t + m * step, ones_col)
            q = q + (cnt >= kf).astype(jnp.int32)
        return t + q * step

    t = lax.fori_loop(0, 8, vbody, t0)

    sel_gt = skey > t
    cnt_gt = _lane_count(sel_gt, ones_col)
    need = kf - cnt_gt
    eq = skey == t

    idx = lax.broadcasted_iota(jnp.int32, (_B, _V), 1)

    def ibody(i, cacc):
        bit = (jnp.int32(8) - 4 * i).astype(jnp.int32)
        step = lax.shift_left(jnp.int32(1), bit)
        q = jnp.zeros((_B, 1), jnp.int32)
        for m in range(1, 16):
            cnt = _lane_count(eq & (idx < cacc + m * step), ones_col)
            q = q + (cnt < need).astype(jnp.int32)
        return cacc + q * step

    c = lax.fori_loop(0, 3, ibody, jnp.zeros((_B, 1), jnp.int32))

    sel = sel_gt | (eq & (idx <= c))
    actions = sel.astype(jnp.float32)
    actions_ref[...] = actions
    logprobs_ref[...] = logp * actions


def _tc_head(counts, embt, attention_mask, w, b_cls, gumbel):
    return pl.pallas_call(
        _tc_head_body,
        out_shape=(
            jax.ShapeDtypeStruct((1, _B), jnp.float32),
            jax.ShapeDtypeStruct((_B, _V), jnp.float32),
            jax.ShapeDtypeStruct((_B, _V), jnp.float32),
        ),
    )(counts, embt, attention_mask, w, b_cls, gumbel)


def kernel(input_ids, attention_mask, emb_table, W_cls, b_cls, gumbel_noise):
    counts = _sc_hist()(input_ids.astype(jnp.int32))
    vals, logprobs, actions = _tc_head(
        counts, emb_table.T.astype(jnp.bfloat16),
        attention_mask.astype(jnp.int32), W_cls,
        b_cls.reshape(1, _V), gumbel_noise)
    return (vals.reshape(_B), logprobs, actions)


# speedup vs baseline: 1.3437x; 1.3437x over previous
import functools

import jax
import jax.numpy as jnp
from jax import lax
from jax.experimental import pallas as pl
from jax.experimental.pallas import tpu as pltpu
from jax.experimental.pallas import tpu_sc as plsc

_B = 8
_S = 2048
_V = 2048
_D = 64
_K = 1000
_TOK = 32000


def _sc_hist_body(ids_hbm, out_hbm, idx_v, hist_v):
    c = lax.axis_index("c")
    s = lax.axis_index("s")
    b = s * 2 + c

    @pl.when(b < _B)
    def _():
        pltpu.sync_copy(ids_hbm.at[b], idx_v)
        zeros16 = jnp.zeros((16,), jnp.float32)

        def zbody(i, _):
            base = i * 256
            for j in range(16):
                hist_v[pl.ds(base + j * 16, 16)] = zeros16
            return 0

        lax.fori_loop(0, _TOK // 256, zbody, 0)
        ones16 = jnp.ones((16,), jnp.float32)

        def sbody(k, _):
            base = k * 64
            for j in range(4):
                ids16 = idx_v[pl.ds(base + j * 16, 16)]
                plsc.addupdate_scatter(hist_v, [ids16], ones16)
            return 0

        lax.fori_loop(0, _S // 64, sbody, 0)
        pltpu.sync_copy(hist_v, out_hbm.at[b])


@functools.cache
def _sc_hist():
    return pl.kernel(
        _sc_hist_body,
        out_type=jax.ShapeDtypeStruct((_B, _TOK), jnp.float32),
        mesh=plsc.VectorSubcoreMesh(core_axis_name="c", subcore_axis_name="s"),
        scratch_types=[
            pltpu.VMEM((_S,), jnp.int32),
            pltpu.VMEM((_TOK,), jnp.float32),
        ],
        compiler_params=pltpu.CompilerParams(needs_layout_passes=False,
                                             skip_device_barrier=True),
    )


def _lane_count(mask_bool, ones_col):
    del ones_col
    return jnp.sum(mask_bool.astype(jnp.float32), axis=1, keepdims=True)


def _tc_head_body(counts_ref, embt_ref, mask_ref, w_ref, b_ref, gum_ref,
                  values_ref, logprobs_ref, actions_ref):
    psum = lax.dot_general(
        counts_ref[...].astype(jnp.bfloat16), embt_ref[...],
        (((1,), (1,)), ((), ())),
        preferred_element_type=jnp.float32,
    )
    mask = mask_ref[...].astype(jnp.float32)
    denom = jnp.maximum(jnp.sum(mask, axis=1, keepdims=True), 1e-6)
    pooled = psum / denom

    logits = jnp.dot(pooled, w_ref[...],
                     preferred_element_type=jnp.float32,
                     precision=lax.Precision.HIGHEST) + b_ref[...]

    rowmax = jnp.max(logits, axis=1, keepdims=True)
    values_ref[...] = jnp.transpose(jax.nn.sigmoid(rowmax))

    shifted = logits - rowmax
    lse = jnp.log(jnp.sum(jnp.exp(shifted), axis=1, keepdims=True))
    logp = shifted - lse

    g0 = logits + gum_ref[...]
    s = lax.bitcast_convert_type(g0, jnp.int32)
    skey = jnp.where(s >= 0, s, s ^ jnp.int32(0x7FFFFFFF))

    ones_col = jnp.ones((_V, 1), jnp.bfloat16)

    t0 = jnp.full((_B, 1), jnp.int32(-2147483648))
    kf = jnp.float32(_K)

    def vbody(i, t):
        bit = (jnp.int32(28) - 4 * i).astype(jnp.int32)
        step = lax.shift_left(jnp.int32(1), bit)
        q = jnp.zeros((_B, 1), jnp.int32)
        for m in range(1, 16):
            cnt = _lane_count(skey >= t + m * step, ones_col)
            q = q + (cnt >= kf).astype(jnp.int32)
        return t + q * step

    t = lax.fori_loop(0, 8, vbody, t0)

    sel_gt = skey > t
    cnt_gt = _lane_count(sel_gt, ones_col)
    need = kf - cnt_gt
    eq = skey == t

    idx = lax.broadcasted_iota(jnp.int32, (_B, _V), 1)

    def ibody(i, cacc):
        bit = (jnp.int32(8) - 4 * i).astype(jnp.int32)
        step = lax.shift_left(jnp.int32(1), bit)
        q = jnp.zeros((_B, 1), jnp.int32)
        for m in range(1, 16):
            cnt = _lane_count(eq & (idx < cacc + m * step), ones_col)
            q = q + (cnt < need).astype(jnp.int32)
        return cacc + q * step

    c = lax.fori_loop(0, 3, ibody, jnp.zeros((_B, 1), jnp.int32))

    sel = sel_gt | (eq & (idx <= c))
    actions = sel.astype(jnp.float32)
    actions_ref[...] = actions
    logprobs_ref[...] = logp * actions


def _tc_head(counts, embt, attention_mask, w, b_cls, gumbel):
    return pl.pallas_call(
        _tc_head_body,
        out_shape=(
            jax.ShapeDtypeStruct((1, _B), jnp.float32),
            jax.ShapeDtypeStruct((_B, _V), jnp.float32),
            jax.ShapeDtypeStruct((_B, _V), jnp.float32),
        ),
    )(counts, embt, attention_mask, w, b_cls, gumbel)


def kernel(input_ids, attention_mask, emb_table, W_cls, b_cls, gumbel_noise):
    counts = _sc_hist()(input_ids.astype(jnp.int32))
    vals, logprobs, actions = _tc_head(
        counts, emb_table.T.astype(jnp.bfloat16),
        attention_mask.astype(jnp.int32), W_cls,
        b_cls.reshape(1, _V), gumbel_noise)
    return (vals.reshape(_B), logprobs, actions)


# speedup vs baseline: 1.3814x; 1.0280x over previous
import functools

import jax
import jax.numpy as jnp
from jax import lax
from jax.experimental import pallas as pl
from jax.experimental.pallas import tpu as pltpu
from jax.experimental.pallas import tpu_sc as plsc

_B = 8
_S = 2048
_V = 2048
_D = 64
_K = 1000
_TOK = 32000


def _sc_hist_body(ids_hbm, out_hbm, idx_v, hist_v):
    c = lax.axis_index("c")
    s = lax.axis_index("s")
    b = s * 2 + c

    @pl.when(b < _B)
    def _():
        pltpu.sync_copy(ids_hbm.at[b], idx_v)
        zeros16 = jnp.zeros((16,), jnp.float32)

        def zbody(i, _):
            base = i * 256
            for j in range(16):
                hist_v[pl.ds(base + j * 16, 16)] = zeros16
            return 0

        lax.fori_loop(0, _TOK // 256, zbody, 0)
        ones16 = jnp.ones((16,), jnp.float32)

        def sbody(k, _):
            base = k * 64
            for j in range(4):
                ids16 = idx_v[pl.ds(base + j * 16, 16)]
                plsc.addupdate_scatter(hist_v, [ids16], ones16)
            return 0

        lax.fori_loop(0, _S // 64, sbody, 0)
        pltpu.sync_copy(hist_v, out_hbm.at[b])


@functools.cache
def _sc_hist():
    return pl.kernel(
        _sc_hist_body,
        out_type=jax.ShapeDtypeStruct((_B, _TOK), jnp.float32),
        mesh=plsc.VectorSubcoreMesh(core_axis_name="c", subcore_axis_name="s"),
        scratch_types=[
            pltpu.VMEM((_S,), jnp.int32),
            pltpu.VMEM((_TOK,), jnp.float32),
        ],
        compiler_params=pltpu.CompilerParams(needs_layout_passes=False,
                                             skip_device_barrier=True),
    )


def _lane_count(mask_bool, ones_col):
    del ones_col
    return jnp.sum(mask_bool.astype(jnp.float32), axis=1, keepdims=True)


def _tc_head_body(counts_ref, embt_ref, mask_ref, w_ref, b_ref, gum_ref,
                  values_ref, logprobs_ref, actions_ref):
    psum = lax.dot_general(
        counts_ref[...].astype(jnp.bfloat16), embt_ref[...],
        (((1,), (1,)), ((), ())),
        preferred_element_type=jnp.float32,
    )
    mask = mask_ref[...].astype(jnp.float32)
    denom = jnp.maximum(jnp.sum(mask, axis=1, keepdims=True), 1e-6)
    pooled = psum / denom

    logits = jnp.dot(pooled, w_ref[...],
                     preferred_element_type=jnp.float32,
                     precision=lax.Precision.HIGHEST) + b_ref[...]

    rowmax = jnp.max(logits, axis=1, keepdims=True)
    values_ref[...] = jnp.transpose(jax.nn.sigmoid(rowmax))

    shifted = logits - rowmax
    lse = jnp.log(jnp.sum(jnp.exp(shifted), axis=1, keepdims=True))
    logp = shifted - lse

    g0 = logits + gum_ref[...]
    s = lax.bitcast_convert_type(g0, jnp.int32)
    skey = jnp.where(s >= 0, s, s ^ jnp.int32(0x7FFFFFFF))

    ones_col = jnp.ones((_V, 1), jnp.bfloat16)

    t0 = jnp.full((_B, 1), jnp.int32(-2147483648))
    kf = jnp.float32(_K)

    def vbody(i, t):
        bit = (jnp.int32(28) - 4 * i).astype(jnp.int32)
        step = lax.shift_left(jnp.int32(1), bit)
        q = jnp.zeros((_B, 1), jnp.int32)
        for m in range(1, 16):
            cnt = _lane_count(skey >= t + m * step, ones_col)
            q = q + (cnt >= kf).astype(jnp.int32)
        return t + q * step

    t = lax.fori_loop(0, 8, vbody, t0)

    sel_gt = skey > t
    cnt_gt = _lane_count(sel_gt, ones_col)
    need = kf - cnt_gt
    eq = skey == t
    cnt_eq = _lane_count(eq, ones_col)

    idx = lax.broadcasted_iota(jnp.int32, (_B, _V), 1)

    def tie_break(_):
        def ibody(i, cacc):
            bit = (jnp.int32(8) - 4 * i).astype(jnp.int32)
            step = lax.shift_left(jnp.int32(1), bit)
            q = jnp.zeros((_B, 1), jnp.int32)
            for m in range(1, 16):
                cnt = _lane_count(eq & (idx < cacc + m * step), ones_col)
                q = q + (cnt < need).astype(jnp.int32)
            return cacc + q * step

        return lax.fori_loop(0, 3, ibody, jnp.zeros((_B, 1), jnp.int32))

    any_surplus = jnp.any(cnt_eq > need)
    c = lax.cond(any_surplus, tie_break,
                 lambda _: jnp.full((_B, 1), jnp.int32(_V)), None)

    sel = sel_gt | (eq & (idx <= c))
    actions = sel.astype(jnp.float32)
    actions_ref[...] = actions
    logprobs_ref[...] = logp * actions


def _tc_head(counts, embt, attention_mask, w, b_cls, gumbel):
    return pl.pallas_call(
        _tc_head_body,
        out_shape=(
            jax.ShapeDtypeStruct((1, _B), jnp.float32),
            jax.ShapeDtypeStruct((_B, _V), jnp.float32),
            jax.ShapeDtypeStruct((_B, _V), jnp.float32),
        ),
    )(counts, embt, attention_mask, w, b_cls, gumbel)


def kernel(input_ids, attention_mask, emb_table, W_cls, b_cls, gumbel_noise):
    counts = _sc_hist()(input_ids.astype(jnp.int32))
    vals, logprobs, actions = _tc_head(
        counts, emb_table.T.astype(jnp.bfloat16),
        attention_mask.astype(jnp.int32), W_cls,
        b_cls.reshape(1, _V), gumbel_noise)
    return (vals.reshape(_B), logprobs, actions)


# speedup vs baseline: 1.4053x; 1.0173x over previous
import functools

import jax
import jax.numpy as jnp
from jax import lax
from jax.experimental import pallas as pl
from jax.experimental.pallas import tpu as pltpu
from jax.experimental.pallas import tpu_sc as plsc

_B = 8
_S = 2048
_V = 2048
_D = 64
_K = 1000
_TOK = 32000


def _sc_hist_body(ids_hbm, out_hbm, idx_v, hist_v, sem):
    c = lax.axis_index("c")
    s = lax.axis_index("s")
    b = s * 2 + c

    @pl.when(b < _B)
    def _():
        cp = pltpu.async_copy(ids_hbm.at[b], idx_v, sem)
        zeros16 = jnp.zeros((16,), jnp.float32)

        def zbody(i, _):
            base = i * 256
            for j in range(16):
                hist_v[pl.ds(base + j * 16, 16)] = zeros16
            return 0

        lax.fori_loop(0, _TOK // 256, zbody, 0)
        cp.wait()
        ones16 = jnp.ones((16,), jnp.float32)

        def sbody(k, _):
            base = k * 64
            for j in range(4):
                ids16 = idx_v[pl.ds(base + j * 16, 16)]
                plsc.addupdate_scatter(hist_v, [ids16], ones16)
            return 0

        lax.fori_loop(0, _S // 64, sbody, 0)
        pltpu.sync_copy(hist_v, out_hbm.at[b])


@functools.cache
def _sc_hist():
    return pl.kernel(
        _sc_hist_body,
        out_type=jax.ShapeDtypeStruct((_B, _TOK), jnp.float32),
        mesh=plsc.VectorSubcoreMesh(core_axis_name="c", subcore_axis_name="s"),
        scratch_types=[
            pltpu.VMEM((_S,), jnp.int32),
            pltpu.VMEM((_TOK,), jnp.float32),
            pltpu.SemaphoreType.DMA,
        ],
        compiler_params=pltpu.CompilerParams(needs_layout_passes=False,
                                             skip_device_barrier=True),
    )


def _lane_count(mask_bool, ones_col):
    del ones_col
    return jnp.sum(mask_bool.astype(jnp.float32), axis=1, keepdims=True)


def _tc_head_body(counts_ref, embt_ref, mask_ref, w_ref, b_ref, gum_ref,
                  values_ref, logprobs_ref, actions_ref):
    psum = lax.dot_general(
        counts_ref[...].astype(jnp.bfloat16), embt_ref[...],
        (((1,), (1,)), ((), ())),
        preferred_element_type=jnp.float32,
    )
    mask = mask_ref[...].astype(jnp.float32)
    denom = jnp.maximum(jnp.sum(mask, axis=1, keepdims=True), 1e-6)
    pooled = psum / denom

    logits = jnp.dot(pooled, w_ref[...],
                     preferred_element_type=jnp.float32,
                     precision=lax.Precision.HIGHEST) + b_ref[...]

    rowmax = jnp.max(logits, axis=1, keepdims=True)
    values_ref[...] = jnp.transpose(jax.nn.sigmoid(rowmax))

    shifted = logits - rowmax
    lse = jnp.log(jnp.sum(jnp.exp(shifted), axis=1, keepdims=True))
    logp = shifted - lse

    g0 = logits + gum_ref[...]
    s = lax.bitcast_convert_type(g0, jnp.int32)
    skey = jnp.where(s >= 0, s, s ^ jnp.int32(0x7FFFFFFF))

    ones_col = jnp.ones((_V, 1), jnp.bfloat16)

    t0 = jnp.full((_B, 1), jnp.int32(-2147483648))
    kf = jnp.float32(_K)

    def vbody(i, t):
        bit = (jnp.int32(28) - 4 * i).astype(jnp.int32)
        step = lax.shift_left(jnp.int32(1), bit)
        q = jnp.zeros((_B, 1), jnp.int32)
        for m in range(1, 16):
            cnt = _lane_count(skey >= t + m * step, ones_col)
            q = q + (cnt >= kf).astype(jnp.int32)
        return t + q * step

    t = lax.fori_loop(0, 8, vbody, t0)

    sel_gt = skey > t
    cnt_gt = _lane_count(sel_gt, ones_col)
    need = kf - cnt_gt
    eq = skey == t
    cnt_eq = _lane_count(eq, ones_col)

    idx = lax.broadcasted_iota(jnp.int32, (_B, _V), 1)

    def tie_break(_):
        def ibody(i, cacc):
            bit = (jnp.int32(8) - 4 * i).astype(jnp.int32)
            step = lax.shift_left(jnp.int32(1), bit)
            q = jnp.zeros((_B, 1), jnp.int32)
            for m in range(1, 16):
                cnt = _lane_count(eq & (idx < cacc + m * step), ones_col)
                q = q + (cnt < need).astype(jnp.int32)
            return cacc + q * step

        return lax.fori_loop(0, 3, ibody, jnp.zeros((_B, 1), jnp.int32))

    any_surplus = jnp.any(cnt_eq > need)
    c = lax.cond(any_surplus, tie_break,
                 lambda _: jnp.full((_B, 1), jnp.int32(_V)), None)

    sel = sel_gt | (eq & (idx <= c))
    actions = sel.astype(jnp.float32)
    actions_ref[...] = actions
    logprobs_ref[...] = logp * actions


def _tc_head(counts, embt, attention_mask, w, b_cls, gumbel):
    return pl.pallas_call(
        _tc_head_body,
        out_shape=(
            jax.ShapeDtypeStruct((1, _B), jnp.float32),
            jax.ShapeDtypeStruct((_B, _V), jnp.float32),
            jax.ShapeDtypeStruct((_B, _V), jnp.float32),
        ),
    )(counts, embt, attention_mask, w, b_cls, gumbel)


def kernel(input_ids, attention_mask, emb_table, W_cls, b_cls, gumbel_noise):
    counts = _sc_hist()(input_ids.astype(jnp.int32))
    vals, logprobs, actions = _tc_head(
        counts, emb_table.T.astype(jnp.bfloat16),
        attention_mask.astype(jnp.int32), W_cls,
        b_cls.reshape(1, _V), gumbel_noise)
    return (vals.reshape(_B), logprobs, actions)


# speedup vs baseline: 1.4245x; 1.0136x over previous
import functools

import jax
import jax.numpy as jnp
from jax import lax
from jax.experimental import pallas as pl
from jax.experimental.pallas import tpu as pltpu
from jax.experimental.pallas import tpu_sc as plsc

_B = 8
_S = 2048
_V = 2048
_D = 64
_K = 1000
_TOK = 32000


def _sc_hist_body(ids_hbm, out_hbm, idx_v, hist_v, sem):
    c = lax.axis_index("c")
    s = lax.axis_index("s")
    b = s * 2 + c

    @pl.when(b < _B)
    def _():
        cp = pltpu.async_copy(ids_hbm.at[b], idx_v, sem)
        zeros16 = jnp.zeros((16,), jnp.float32)

        def zbody(i, _):
            base = i * 256
            for j in range(16):
                hist_v[pl.ds(base + j * 16, 16)] = zeros16
            return 0

        lax.fori_loop(0, _TOK // 256, zbody, 0)
        cp.wait()
        ones16 = jnp.ones((16,), jnp.float32)

        def sbody(k, _):
            base = k * 64
            for j in range(4):
                ids16 = idx_v[pl.ds(base + j * 16, 16)]
                plsc.addupdate_scatter(hist_v, [ids16], ones16)
            return 0

        lax.fori_loop(0, _S // 64, sbody, 0)
        pltpu.sync_copy(hist_v, out_hbm.at[b])


@functools.cache
def _sc_hist():
    return pl.kernel(
        _sc_hist_body,
        out_type=jax.ShapeDtypeStruct((_B, _TOK), jnp.float32),
        mesh=plsc.VectorSubcoreMesh(core_axis_name="c", subcore_axis_name="s"),
        scratch_types=[
            pltpu.VMEM((_S,), jnp.int32),
            pltpu.VMEM((_TOK,), jnp.float32),
            pltpu.SemaphoreType.DMA,
        ],
        compiler_params=pltpu.CompilerParams(needs_layout_passes=False,
                                             skip_device_barrier=True),
    )


def _lane_count(mask_bool):
    return jnp.sum(mask_bool.astype(jnp.float32), axis=1, keepdims=True)


def _tc_head_body(counts_ref, embt_ref, mask_ref, w_ref, b_ref, gum_ref,
                  values_ref, logprobs_ref, actions_ref):
    psum = lax.dot_general(
        counts_ref[...].astype(jnp.bfloat16), embt_ref[...],
        (((1,), (1,)), ((), ())),
        preferred_element_type=jnp.float32,
    )
    mask = mask_ref[...].astype(jnp.float32)
    denom = jnp.maximum(jnp.sum(mask, axis=1, keepdims=True), 1e-6)
    pooled = psum / denom

    logits = jnp.dot(pooled, w_ref[...],
                     preferred_element_type=jnp.float32,
                     precision=lax.Precision.HIGHEST) + b_ref[...]

    rowmax = jnp.max(logits, axis=1, keepdims=True)
    values_ref[...] = jnp.transpose(jax.nn.sigmoid(rowmax))

    shifted = logits - rowmax
    lse = jnp.log(jnp.sum(jnp.exp(shifted), axis=1, keepdims=True))
    logp = shifted - lse

    g0 = logits + gum_ref[...]
    s = lax.bitcast_convert_type(g0, jnp.int32)
    skey = jnp.where(s >= 0, s, s ^ jnp.int32(0x7FFFFFFF))

    t0 = jnp.full((_B, 1), jnp.int32(-2147483648))
    kf = jnp.float32(_K)

    def vbody(i, t):
        bit = (jnp.int32(28) - 4 * i).astype(jnp.int32)
        step = lax.shift_left(jnp.int32(1), bit)
        q = jnp.zeros((_B, 1), jnp.int32)
        for m in range(1, 16):
            cnt = _lane_count(skey >= t + m * step)
            q = q + (cnt >= kf).astype(jnp.int32)
        return t + q * step

    t = lax.fori_loop(0, 8, vbody, t0)

    sel_gt = skey > t
    cnt_gt = _lane_count(sel_gt)
    need = kf - cnt_gt
    eq = skey == t
    cnt_eq = _lane_count(eq)

    idx = lax.broadcasted_iota(jnp.int32, (_B, _V), 1)

    def tie_break(_):
        def ibody(i, cacc):
            bit = (jnp.int32(8) - 4 * i).astype(jnp.int32)
            step = lax.shift_left(jnp.int32(1), bit)
            q = jnp.zeros((_B, 1), jnp.int32)
            for m in range(1, 16):
                cnt = _lane_count(eq & (idx < cacc + m * step))
                q = q + (cnt < need).astype(jnp.int32)
            return cacc + q * step

        return lax.fori_loop(0, 3, ibody, jnp.zeros((_B, 1), jnp.int32))

    any_surplus = jnp.any(cnt_eq > need)
    c = lax.cond(any_surplus, tie_break,
                 lambda _: jnp.full((_B, 1), jnp.int32(_V)), None)

    sel = sel_gt | (eq & (idx <= c))
    actions = sel.astype(jnp.float32)
    actions_ref[...] = actions
    logprobs_ref[...] = logp * actions


def _tc_head(counts, embt, attention_mask, w, b_cls, gumbel):
    return pl.pallas_call(
        _tc_head_body,
        out_shape=(
            jax.ShapeDtypeStruct((1, _B), jnp.float32),
            jax.ShapeDtypeStruct((_B, _V), jnp.float32),
            jax.ShapeDtypeStruct((_B, _V), jnp.float32),
        ),
    )(counts, embt, attention_mask, w, b_cls, gumbel)


def kernel(input_ids, attention_mask, emb_table, W_cls, b_cls, gumbel_noise):
    counts = _sc_hist()(input_ids.astype(jnp.int32))
    vals, logprobs, actions = _tc_head(
        counts, emb_table.T.astype(jnp.bfloat16),
        attention_mask.astype(jnp.int32), W_cls,
        b_cls.reshape(1, _V), gumbel_noise)
    return (vals.reshape(_B), logprobs, actions)
